# Initial kernel scaffold; baseline (speedup 1.0000x reference)
import jax
import jax.numpy as jnp
from jax.experimental import pallas as pl


def kernel(x, conv1_w, conv1_scale, conv1_shift, L0b0_c1_w, L0b0_c1_scale, L0b0_c1_shift, L0b0_c2_w, L0b0_c2_scale, L0b0_c2_shift, L0b0_c3_w, L0b0_c3_scale, L0b0_c3_shift, L0b0_down_w, L0b0_down_scale, L0b0_down_shift, L0b1_c1_w, L0b1_c1_scale, L0b1_c1_shift, L0b1_c2_w, L0b1_c2_scale, L0b1_c2_shift, L0b1_c3_w, L0b1_c3_scale, L0b1_c3_shift, L0b2_c1_w, L0b2_c1_scale, L0b2_c1_shift, L0b2_c2_w, L0b2_c2_scale, L0b2_c2_shift, L0b2_c3_w, L0b2_c3_scale, L0b2_c3_shift, L1b0_c1_w, L1b0_c1_scale, L1b0_c1_shift, L1b0_c2_w, L1b0_c2_scale, L1b0_c2_shift, L1b0_c3_w, L1b0_c3_scale, L1b0_c3_shift, L1b0_down_w, L1b0_down_scale, L1b0_down_shift, L1b1_c1_w, L1b1_c1_scale, L1b1_c1_shift, L1b1_c2_w, L1b1_c2_scale, L1b1_c2_shift, L1b1_c3_w, L1b1_c3_scale, L1b1_c3_shift, L1b2_c1_w, L1b2_c1_scale, L1b2_c1_shift, L1b2_c2_w, L1b2_c2_scale, L1b2_c2_shift, L1b2_c3_w, L1b2_c3_scale, L1b2_c3_shift, L1b3_c1_w, L1b3_c1_scale, L1b3_c1_shift, L1b3_c2_w, L1b3_c2_scale, L1b3_c2_shift, L1b3_c3_w, L1b3_c3_scale, L1b3_c3_shift, L2b0_c1_w, L2b0_c1_scale, L2b0_c1_shift, L2b0_c2_w, L2b0_c2_scale, L2b0_c2_shift, L2b0_c3_w, L2b0_c3_scale, L2b0_c3_shift, L2b0_down_w, L2b0_down_scale, L2b0_down_shift, L2b1_c1_w, L2b1_c1_scale, L2b1_c1_shift, L2b1_c2_w, L2b1_c2_scale, L2b1_c2_shift, L2b1_c3_w, L2b1_c3_scale, L2b1_c3_shift, L2b2_c1_w, L2b2_c1_scale, L2b2_c1_shift, L2b2_c2_w, L2b2_c2_scale, L2b2_c2_shift, L2b2_c3_w, L2b2_c3_scale, L2b2_c3_shift, L2b3_c1_w, L2b3_c1_scale, L2b3_c1_shift, L2b3_c2_w, L2b3_c2_scale, L2b3_c2_shift, L2b3_c3_w, L2b3_c3_scale, L2b3_c3_shift, L2b4_c1_w, L2b4_c1_scale, L2b4_c1_shift, L2b4_c2_w, L2b4_c2_scale, L2b4_c2_shift, L2b4_c3_w, L2b4_c3_scale, L2b4_c3_shift, L2b5_c1_w, L2b5_c1_scale, L2b5_c1_shift, L2b5_c2_w, L2b5_c2_scale, L2b5_c2_shift, L2b5_c3_w, L2b5_c3_scale, L2b5_c3_shift, L3b0_c1_w, L3b0_c1_scale, L3b0_c1_shift, L3b0_c2_w, L3b0_c2_scale, L3b0_c2_shift, L3b0_c3_w, L3b0_c3_scale, L3b0_c3_shift, L3b0_down_w, L3b0_down_scale, L3b0_down_shift, L3b1_c1_w, L3b1_c1_scale, L3b1_c1_shift, L3b1_c2_w, L3b1_c2_scale, L3b1_c2_shift, L3b1_c3_w, L3b1_c3_scale, L3b1_c3_shift, L3b2_c1_w, L3b2_c1_scale, L3b2_c1_shift, L3b2_c2_w, L3b2_c2_scale, L3b2_c2_shift, L3b2_c3_w, L3b2_c3_scale, L3b2_c3_shift, fc_w, fc_scale, fc_shift):
    raise NotImplementedError("write your pallas kernel here")


# speedup vs baseline: 1.2634x; 1.2634x over previous
import functools

import jax
import jax.numpy as jnp
from jax.experimental import pallas as pl
from jax.experimental.pallas import tpu as pltpu


def _nbytes(shape, dtype):
    n = 1
    for d in shape:
        n *= int(d)
    return n * jnp.dtype(dtype).itemsize


def _vlim(est_bytes):
    est = int(2.4 * est_bytes) + (4 << 20)
    return min(max(est, 16 * 1024 * 1024), 56 * 1024 * 1024)


def _stem_kernel(a_ref, w_ref, s_ref, t_ref, o_ref, pp_ref, *, Nh, Ho, Wo):
    y = jnp.dot(a_ref[...], w_ref[...], preferred_element_type=jnp.float32)
    y = jnp.maximum(y * s_ref[...] + t_ref[...], 0.0).astype(jnp.bfloat16)
    C = y.shape[-1]
    Hp, Wp = Ho // 2, Wo // 2
    pp_ref[...] = jnp.zeros(pp_ref.shape, pp_ref.dtype)
    pp_ref[:, 1:Ho + 1, 1:Wo + 1, :] = y.reshape(Nh, Ho, Wo, C)
    rm = jnp.maximum(jnp.maximum(pp_ref[:, 0:Ho, :, :], pp_ref[:, 1:Ho + 1, :, :]),
                     pp_ref[:, 2:Ho + 2, :, :])
    re = rm.reshape(Nh, Hp, 2, Wo + 2, C)[:, :, 0]
    cm = jnp.maximum(jnp.maximum(re[:, :, 0:Wo, :], re[:, :, 1:Wo + 1, :]),
                     re[:, :, 2:Wo + 2, :])
    ce = cm.reshape(Nh, Hp, Wp, 2, C)[:, :, :, 0]
    o_ref[...] = ce.reshape(Nh * Hp * Wp, C)


def _stem_call(a, w, s, t, N, Ho, Wo):
    M, K = a.shape
    C = w.shape[1]
    Nh = N // 2
    Mh = M // 2
    Hp, Wp = Ho // 2, Wo // 2
    Mo = N * Hp * Wp
    est = (_nbytes((Mh, K), jnp.bfloat16) + _nbytes((K, C), jnp.bfloat16)
           + _nbytes((Nh, Ho + 2, Wo + 2, C), jnp.bfloat16)
           + _nbytes((Mo // 2, C), jnp.bfloat16))
    return pl.pallas_call(
        functools.partial(_stem_kernel, Nh=Nh, Ho=Ho, Wo=Wo),
        out_shape=jax.ShapeDtypeStruct((Mo, C), jnp.bfloat16),
        grid_spec=pltpu.PrefetchScalarGridSpec(
            num_scalar_prefetch=0,
            grid=(2,),
            in_specs=[
                pl.BlockSpec((Mh, K), lambda c: (c, 0)),
                pl.BlockSpec((K, C), lambda c: (0, 0)),
                pl.BlockSpec((1, C), lambda c: (0, 0)),
                pl.BlockSpec((1, C), lambda c: (0, 0)),
            ],
            out_specs=pl.BlockSpec((Mo // 2, C), lambda c: (c, 0)),
            scratch_shapes=[pltpu.VMEM((Nh, Ho + 2, Wo + 2, C), jnp.bfloat16)],
        ),
        compiler_params=pltpu.CompilerParams(
            dimension_semantics=("parallel",),
            vmem_limit_bytes=_vlim(est),
        ),
    )(a, w, s, t)


def _block_kernel(*refs, Nh, H, W, P, Cin, stride, has_down, use_col):
    x_ref = refs[0]
    w1, s1, t1, w2, s2, t2, w3, s3, t3 = refs[1:10]
    pos = 10
    if has_down:
        wd, sd, td = refs[pos:pos + 3]
        pos += 3
    o_ref = refs[pos]
    pad_ref = refs[pos + 1]
    col_ref = refs[pos + 2] if use_col else None

    Ho, Wo = H // stride, W // stride
    xv = x_ref[...]
    y1 = jnp.dot(xv, w1[...], preferred_element_type=jnp.float32)
    y1 = jnp.maximum(y1 * s1[...] + t1[...], 0.0).astype(jnp.bfloat16)

    pad_ref[...] = jnp.zeros(pad_ref.shape, pad_ref.dtype)
    pad_ref[:, 1:H + 1, 1:W + 1, :] = y1.reshape(Nh, H, W, P)
    taps = []
    for di in range(3):
        for dj in range(3):
            tap = pad_ref[:, di:di + H, dj:dj + W, :]
            if stride == 2:
                tap = tap.reshape(Nh, Ho, 2, Wo, 2, P)[:, :, 0, :, 0, :]
            taps.append(tap.reshape(Nh * Ho * Wo, P))
    if use_col:
        for ti, tp in enumerate(taps):
            col_ref[:, ti * P:(ti + 1) * P] = tp
        y2 = jnp.dot(col_ref[...], w2[0:9 * P, :],
                     preferred_element_type=jnp.float32)
    else:
        y2 = None
        for ti, tp in enumerate(taps):
            d = jnp.dot(tp, w2[ti * P:(ti + 1) * P, :],
                        preferred_element_type=jnp.float32)
            y2 = d if y2 is None else y2 + d
    y2 = jnp.maximum(y2 * s2[...] + t2[...], 0.0).astype(jnp.bfloat16)

    y3 = jnp.dot(y2, w3[...], preferred_element_type=jnp.float32)
    y3 = y3 * s3[...] + t3[...]

    if has_down:
        if stride == 2:
            xd = xv.reshape(Nh, Ho, 2, Wo, 2, Cin)[:, :, 0, :, 0, :]
            xd = xd.reshape(Nh * Ho * Wo, Cin)
        else:
            xd = xv
        r = jnp.dot(xd, wd[...], preferred_element_type=jnp.float32)
        r = r * sd[...] + td[...]
    else:
        r = xv.astype(jnp.float32)
    o_ref[...] = jnp.maximum(y3 + r, 0.0).astype(jnp.bfloat16)


def _block_call(h, c1, c2, c3, down, stride, N, H, W):
    M, Cin = h.shape
    P = c1[0].shape[1]
    Kp = c2[0].shape[0]
    Cout = c3[0].shape[1]
    Nh = N // 2
    Mh = M // 2
    Ho, Wo = H // stride, W // stride
    Mho = N * Ho * Wo // 2
    has_down = down is not None
    use_col = (P % 128 == 0)

    args = [h, c1[0], c1[1], c1[2], c2[0], c2[1], c2[2], c3[0], c3[1], c3[2]]
    in_specs = [
        pl.BlockSpec((Mh, Cin), lambda c: (c, 0)),
        pl.BlockSpec((Cin, P), lambda c: (0, 0)),
        pl.BlockSpec((1, P), lambda c: (0, 0)),
        pl.BlockSpec((1, P), lambda c: (0, 0)),
        pl.BlockSpec((Kp, P), lambda c: (0, 0)),
        pl.BlockSpec((1, P), lambda c: (0, 0)),
        pl.BlockSpec((1, P), lambda c: (0, 0)),
        pl.BlockSpec((P, Cout), lambda c: (0, 0)),
        pl.BlockSpec((1, Cout), lambda c: (0, 0)),
        pl.BlockSpec((1, Cout), lambda c: (0, 0)),
    ]
    if has_down:
        args += [down[0], down[1], down[2]]
        in_specs += [
            pl.BlockSpec((Cin, Cout), lambda c: (0, 0)),
            pl.BlockSpec((1, Cout), lambda c: (0, 0)),
            pl.BlockSpec((1, Cout), lambda c: (0, 0)),
        ]

    scratch = [pltpu.VMEM((Nh, H + 2, W + 2, P), jnp.bfloat16)]
    scr = _nbytes((Nh, H + 2, W + 2, P), jnp.bfloat16)
    if use_col:
        scratch.append(pltpu.VMEM((Mho, 9 * P), jnp.bfloat16))
        scr += _nbytes((Mho, 9 * P), jnp.bfloat16)

    est = (_nbytes((Mh, Cin), jnp.bfloat16) + _nbytes((Cin, P), jnp.bfloat16)
           + _nbytes((Kp, P), jnp.bfloat16) + _nbytes((P, Cout), jnp.bfloat16)
           + (_nbytes((Cin, Cout), jnp.bfloat16) if has_down else 0)
           + _nbytes((Mho, Cout), jnp.bfloat16) + scr)

    out = pl.pallas_call(
        functools.partial(_block_kernel, Nh=Nh, H=H, W=W, P=P, Cin=Cin,
                          stride=stride, has_down=has_down, use_col=use_col),
        out_shape=jax.ShapeDtypeStruct((N * Ho * Wo, Cout), jnp.bfloat16),
        grid_spec=pltpu.PrefetchScalarGridSpec(
            num_scalar_prefetch=0,
            grid=(2,),
            in_specs=in_specs,
            out_specs=pl.BlockSpec((Mho, Cout), lambda c: (c, 0)),
            scratch_shapes=scratch,
        ),
        compiler_params=pltpu.CompilerParams(
            dimension_semantics=("parallel",),
            vmem_limit_bytes=_vlim(est),
        ),
    )(*args)
    return out


def _fc_kernel(x_ref, w_ref, s_ref, t_ref, o_ref, *, N, HW):
    feat = x_ref[...].astype(jnp.float32).reshape(N, HW, x_ref.shape[-1])
    feat = feat.mean(axis=1)
    y = jnp.dot(feat.astype(jnp.bfloat16), w_ref[...],
                preferred_element_type=jnp.float32)
    y = y * s_ref[...] + t_ref[...]
    o_ref[...] = jax.nn.sigmoid(y)


def _fc_call(h, w, s, t, N, HW):
    M, C = h.shape
    Cout = w.shape[1]
    est = (_nbytes((M, C), jnp.bfloat16) + _nbytes(w.shape, jnp.bfloat16)
           + _nbytes((N, Cout), jnp.float32))
    return pl.pallas_call(
        functools.partial(_fc_kernel, N=N, HW=HW),
        out_shape=jax.ShapeDtypeStruct((N, Cout), jnp.float32),
        grid_spec=pltpu.PrefetchScalarGridSpec(
            num_scalar_prefetch=0,
            grid=(1,),
            in_specs=[
                pl.BlockSpec((M, C), lambda i: (0, 0)),
                pl.BlockSpec((C, Cout), lambda i: (0, 0)),
                pl.BlockSpec((1, Cout), lambda i: (0, 0)),
                pl.BlockSpec((1, Cout), lambda i: (0, 0)),
            ],
            out_specs=pl.BlockSpec((N, Cout), lambda i: (0, 0)),
        ),
        compiler_params=pltpu.CompilerParams(
            dimension_semantics=("arbitrary",),
            vmem_limit_bytes=_vlim(est),
        ),
    )(h, w, s, t)


def _stem_patches(x, Kp):
    xh = jnp.transpose(x, (0, 2, 3, 1)).astype(jnp.bfloat16)
    N, H, W, C = xh.shape
    xp = jnp.pad(xh, ((0, 0), (3, 3), (3, 3), (0, 0)))
    Ho, Wo = H // 2, W // 2
    taps = [xp[:, i:i + 2 * Ho:2, j:j + 2 * Wo:2, :]
            for i in range(7) for j in range(7)]
    a = jnp.stack(taps, axis=3).reshape(N * Ho * Wo, 49 * C)
    if a.shape[1] < Kp:
        a = jnp.pad(a, ((0, 0), (0, Kp - a.shape[1])))
    return a, Ho, Wo


def kernel(*args):
    a = list(args)
    x = a[0]
    conv1 = a[1:4]
    idx = 4
    nblocks = [3, 4, 6, 3]
    layers = []
    for L in range(4):
        blocks = []
        for b in range(nblocks[L]):
            c1 = a[idx:idx + 3]
            c2 = a[idx + 3:idx + 6]
            c3 = a[idx + 6:idx + 9]
            idx += 9
            down = None
            if b == 0:
                down = a[idx:idx + 3]
                idx += 3
            stride = 2 if (L > 0 and b == 0) else 1
            blocks.append((c1, c2, c3, down, stride))
        layers.append(blocks)
    fc_w, fc_scale, fc_shift = a[idx:idx + 3]

    N = x.shape[0]
    patches, Ho, Wo = _stem_patches(x, conv1[0].shape[0])
    h = _stem_call(patches, conv1[0], conv1[1], conv1[2], N, Ho, Wo)
    H = W = Ho // 2
    for blocks in layers:
        for (c1, c2, c3, down, stride) in blocks:
            h = _block_call(h, c1, c2, c3, down, stride, N, H, W)
            H, W = H // stride, W // stride
    out = _fc_call(h, fc_w, fc_scale, fc_shift, N, H * W)
    return out[:, :1]
